# baseline (speedup 1.0000x reference)
import functools

import jax
import jax.numpy as jnp
from jax import lax
from jax.experimental import pallas as pl
from jax.experimental.pallas import tpu as pltpu
from jax.experimental.pallas import tpu_sc as plsc

NUM_EMB = 1000000
DIM = 32
QMAX = 255.0
MAGIC = float(1.5 * 2**23)

ROWS_PER_STEP = 8000
MINMAX_GRID = NUM_EMB // ROWS_PER_STEP

def _tc_body(w_ref, m_ref, scale_ref, zp_ref, mp_ref, mn_ref, mx_ref):
    i = pl.program_id(0)

    @pl.when(i == 0)
    def _init():
        mn_ref[0] = 0.0
        mx_ref[0] = 0.0

    w = w_ref[...]
    mn_ref[0] = jnp.minimum(mn_ref[0], jnp.min(w))
    mx_ref[0] = jnp.maximum(mx_ref[0], jnp.max(w))

    e = lax.broadcasted_iota(jnp.int32, (DIM, 8), 0)
    h = lax.broadcasted_iota(jnp.int32, (DIM, 8), 1)
    pack = jnp.where((e >> 4) == h, 1 << (e & 15), 0).astype(jnp.float32)
    mf = m_ref[...].astype(jnp.float32)
    mp_ref[...] = jax.lax.dot(
        mf, pack,
        precision=jax.lax.Precision.HIGHEST,
        preferred_element_type=jnp.float32)

    @pl.when(i == MINMAX_GRID - 1)
    def _finish():
        mn = mn_ref[0]
        mx = mx_ref[0]
        scale = jnp.maximum((mx - mn) / QMAX, 1e-8)
        zp = jnp.clip(jnp.round(-mn / scale), 0.0, QMAX)
        scale_ref[...] = jnp.full((1, 128), scale, jnp.float32)
        zp_ref[...] = jnp.full((1, 128), zp, jnp.float32)


_scan_pack = pl.pallas_call(
    _tc_body,
    grid=(MINMAX_GRID,),
    in_specs=[
        pl.BlockSpec((ROWS_PER_STEP, DIM), lambda i: (i, 0)),
        pl.BlockSpec((ROWS_PER_STEP, DIM), lambda i: (i, 0)),
    ],
    out_specs=[
        pl.BlockSpec((1, 128), lambda i: (0, 0)),
        pl.BlockSpec((1, 128), lambda i: (0, 0)),
        pl.BlockSpec((ROWS_PER_STEP, 8), lambda i: (i, 0)),
    ],
    out_shape=[
        jax.ShapeDtypeStruct((1, 128), jnp.float32),
        jax.ShapeDtypeStruct((1, 128), jnp.float32),
        jax.ShapeDtypeStruct((NUM_EMB, 8), jnp.float32),
    ],
    scratch_shapes=[
        pltpu.SMEM((1,), jnp.float32),
        pltpu.SMEM((1,), jnp.float32),
    ],
)

B_TOTAL = 4096 * 50
NUM_WORKERS = 32
B_PER_W = B_TOTAL // NUM_WORKERS
CHUNK = 1280
NCHUNK = B_PER_W // CHUNK
SUB = 128
NSUB = CHUNK // SUB

_sc_mesh = plsc.VectorSubcoreMesh(core_axis_name="c", subcore_axis_name="s")


@functools.partial(
    pl.kernel,
    mesh=_sc_mesh,
    out_type=jax.ShapeDtypeStruct((B_TOTAL, DIM), jnp.float32),
    scratch_types=[
        pltpu.VMEM((CHUNK,), jnp.int32),
        pltpu.VMEM((CHUNK, DIM), jnp.float32),
        pltpu.VMEM((CHUNK, 8), jnp.float32),
        pltpu.VMEM((16,), jnp.float32),
        pltpu.VMEM((16,), jnp.float32),
        pltpu.SemaphoreType.DMA,
    ],
    compiler_params=pltpu.CompilerParams(
        needs_layout_passes=False, use_tc_tiling_on_sc=False),
)
def _sc_lookup(idx_hbm, w_hbm, m_hbm, scale_hbm, zp_hbm, out_hbm,
               idx_v, w_v, m_v, scale_v, zp_v, sem):
    wid = lax.axis_index("s") * 2 + lax.axis_index("c")
    base = wid * B_PER_W

    pltpu.sync_copy(scale_hbm.at[pl.ds(0, 16)], scale_v)
    pltpu.sync_copy(zp_hbm.at[pl.ds(0, 16)], zp_v)
    s = scale_v[...]
    zp = zp_v[...]
    inv = 1.0 / s
    lo = -s * zp
    hi = s * (QMAX - zp)
    magic = jnp.full((16,), MAGIC, jnp.float32)
    iota = lax.iota(jnp.int32, 16)

    def do_chunk(c, carry):
        off = base + c * CHUNK
        pltpu.sync_copy(idx_hbm.at[pl.ds(off, CHUNK)], idx_v)
        cps = []
        for sub in range(NSUB):
            isl = idx_v.at[pl.ds(sub * SUB, SUB)]
            cps.append(pltpu.async_copy(
                w_hbm.at[isl], w_v.at[pl.ds(sub * SUB, SUB)], sem))
            cps.append(pltpu.async_copy(
                m_hbm.at[isl], m_v.at[pl.ds(sub * SUB, SUB)], sem))
        for cp in cps:
            cp.wait()

        def do_row(r, carry2):
            rfull = jnp.full((16,), r, jnp.int32)
            for j in range(2):
                w = w_v[r, pl.ds(j * 16, 16)]
                word = plsc.load_gather(
                    m_v, [rfull, jnp.full((16,), j, jnp.int32)])
                bit = (word.astype(jnp.int32) >> iota) & 1
                t = w * inv + zp
                rr = (t + magic) - magic
                q = jnp.clip(rr, 0.0, QMAX)
                wq = (q - zp) * s
                noise = jnp.where(bit == 0, wq - w, 0.0)
                w_v[r, pl.ds(j * 16, 16)] = jnp.clip(w, lo, hi) + noise
            return carry2

        lax.fori_loop(0, CHUNK, do_row, 0)
        pltpu.sync_copy(w_v, out_hbm.at[pl.ds(off, CHUNK)])
        return carry

    lax.fori_loop(0, NCHUNK, do_chunk, 0)


def kernel(input, weight, mask):
    scale_r, zp_r, mpacked = _scan_pack(weight, mask)
    idx = input.reshape(-1)
    out = _sc_lookup(idx, weight, mpacked,
                     scale_r.reshape(-1), zp_r.reshape(-1))
    return out.reshape(input.shape + (DIM,))


# speedup vs baseline: 1.6251x; 1.6251x over previous
import functools

import jax
import jax.numpy as jnp
from jax import lax
from jax.experimental import pallas as pl
from jax.experimental.pallas import tpu as pltpu
from jax.experimental.pallas import tpu_sc as plsc

NUM_EMB = 1000000
DIM = 32
QMAX = 255.0
LINES = NUM_EMB // 4

ROWS_PER_STEP = 2000
TC_GRID = LINES // ROWS_PER_STEP


def _minmax_body(w_ref, scale_ref, zp_ref, mn_ref, mx_ref):
    i = pl.program_id(0)

    @pl.when(i == 0)
    def _init():
        mn_ref[0] = 0.0
        mx_ref[0] = 0.0

    w = w_ref[...]
    mn_ref[0] = jnp.minimum(mn_ref[0], jnp.min(w))
    mx_ref[0] = jnp.maximum(mx_ref[0], jnp.max(w))

    @pl.when(i == TC_GRID - 1)
    def _finish():
        mn = mn_ref[0]
        mx = mx_ref[0]
        scale = jnp.maximum((mx - mn) / QMAX, 1e-8)
        zp = jnp.clip(jnp.round(-mn / scale), 0.0, QMAX)
        scale_ref[...] = jnp.full((1, 128), scale, jnp.float32)
        zp_ref[...] = jnp.full((1, 128), zp, jnp.float32)


_minmax = pl.pallas_call(
    _minmax_body,
    grid=(TC_GRID,),
    in_specs=[pl.BlockSpec((ROWS_PER_STEP, 128), lambda i: (i, 0))],
    out_specs=[
        pl.BlockSpec((1, 128), lambda i: (0, 0)),
        pl.BlockSpec((1, 128), lambda i: (0, 0)),
    ],
    out_shape=[
        jax.ShapeDtypeStruct((1, 128), jnp.float32),
        jax.ShapeDtypeStruct((1, 128), jnp.float32),
    ],
    scratch_shapes=[
        pltpu.SMEM((1,), jnp.float32),
        pltpu.SMEM((1,), jnp.float32),
    ],
)


def _transform_body(w_ref, m_ref, s_ref, z_ref, o_ref):
    s = s_ref[...]
    zp = z_ref[...]
    w = w_ref[...]
    m = m_ref[...]
    t = w / s + zp
    q = jnp.clip(jnp.round(t), 0.0, QMAX)
    wq = (q - zp) * s
    noise = jnp.where(m, 0.0, wq - w)
    o_ref[...] = jnp.clip(w, -s * zp, s * (QMAX - zp)) + noise


_transform = pl.pallas_call(
    _transform_body,
    grid=(TC_GRID,),
    in_specs=[
        pl.BlockSpec((ROWS_PER_STEP, 128), lambda i: (i, 0)),
        pl.BlockSpec((ROWS_PER_STEP, 128), lambda i: (i, 0)),
        pl.BlockSpec((1, 128), lambda i: (0, 0)),
        pl.BlockSpec((1, 128), lambda i: (0, 0)),
    ],
    out_specs=pl.BlockSpec((ROWS_PER_STEP, 128), lambda i: (i, 0)),
    out_shape=jax.ShapeDtypeStruct((LINES, 128), jnp.float32),
)

B_TOTAL = 4096 * 50
NUM_WORKERS = 32
B_PER_W = B_TOTAL // NUM_WORKERS
CHUNK = 640
NCHUNK = B_PER_W // CHUNK
SUB = 128
NSUB = CHUNK // SUB

_sc_mesh = plsc.VectorSubcoreMesh(core_axis_name="c", subcore_axis_name="s")


@functools.partial(
    pl.kernel,
    mesh=_sc_mesh,
    out_type=jax.ShapeDtypeStruct((B_TOTAL // 4, 128), jnp.float32),
    scratch_types=[
        pltpu.VMEM((CHUNK,), jnp.int32),
        pltpu.VMEM((CHUNK,), jnp.int32),
        pltpu.VMEM((CHUNK, 128), jnp.float32),
        pltpu.VMEM((CHUNK // 4, 128), jnp.float32),
        pltpu.SemaphoreType.DMA,
    ],
    compiler_params=pltpu.CompilerParams(needs_layout_passes=False),
)
def _sc_gather(idx_hbm, tab_hbm, out_hbm, idx_v, line_v, g_v, o_v, sem):
    wid = lax.axis_index("s") * 2 + lax.axis_index("c")
    base = wid * B_PER_W

    def do_chunk(c, carry):
        off = pl.multiple_of(base + c * CHUNK, CHUNK)
        pltpu.sync_copy(idx_hbm.at[pl.ds(off, CHUNK)], idx_v)

        def shift_idx(v, carry2):
            line_v[pl.ds(v * 16, 16)] = idx_v[pl.ds(v * 16, 16)] >> 2
            return carry2

        lax.fori_loop(0, CHUNK // 16, shift_idx, 0)
        cps = []
        for sub in range(NSUB):
            cps.append(pltpu.async_copy(
                tab_hbm.at[line_v.at[pl.ds(sub * SUB, SUB)]],
                g_v.at[pl.ds(sub * SUB, SUB)], sem))
        for cp in cps:
            cp.wait()

        def do_group(g, carry2):
            qv = (idx_v[pl.ds(g * 16, 16)] & 3) * 32
            for k in range(16):
                b = g * 16 + k
                orow = g * 4 + (k >> 2)
                for j in range(2):
                    o_v[orow, pl.ds((k & 3) * 32 + j * 16, 16)] = (
                        g_v[b, pl.ds(qv[k] + j * 16, 16)])
            return carry2

        lax.fori_loop(0, CHUNK // 16, do_group, 0)
        pltpu.sync_copy(
            o_v,
            out_hbm.at[pl.ds(pl.multiple_of(off // 4, CHUNK // 4), CHUNK // 4)])
        return carry

    lax.fori_loop(0, NCHUNK, do_chunk, 0)


def kernel(input, weight, mask):
    w4 = weight.reshape(LINES, 128)
    m4 = mask.reshape(LINES, 128)
    scale_r, zp_r = _minmax(w4)
    table = _transform(w4, m4, scale_r, zp_r)
    idx = input.reshape(-1)
    out4 = _sc_gather(idx, table)
    return out4.reshape(input.shape + (DIM,))
